# baseline (speedup 1.0000x reference)
import jax
import jax.numpy as jnp
from jax import lax
from jax.experimental import pallas as pl
from jax.experimental.pallas import tpu as pltpu
from jax.experimental.pallas import tpu_sc as plsc

_B, _C, _N, _K, _DO = 4, 64, 2048, 20, 128
_R = 256
_NB = 128
_NBK = _NB * _K
_NK = _N * _K
_G = 128


def _sq_body(xb_ref, sq_ref):
    xb = xb_ref[0]
    xsq = xb * xb
    sq_ref[0] = jnp.sum(xsq, axis=0, keepdims=True)


def _run_sq(x):
    return pl.pallas_call(
        _sq_body,
        grid=(_B,),
        in_specs=[pl.BlockSpec((1, _C, _N), lambda b: (b, 0, 0))],
        out_specs=pl.BlockSpec((1, 1, _N), lambda b: (b, 0, 0)),
        out_shape=jax.ShapeDtypeStruct((_B, 1, _N), jnp.float32),
    )(x)


def _topk_proj_body(xt_ref, xb_ref, xxr_ref, xxc_ref, awt_ref, bwt_ref,
                    idx_ref, dist_ref, ut_ref, vt_ref, p_scr):
    xt = xt_ref[...]
    xb = xb_ref[...]
    inner = lax.dot_general(xt, xb, (((1,), (0,)), ((), ())),
                            preferred_element_type=jnp.float32)
    p_scr[...] = (2.0 * inner - xxr_ref[...]) - xxc_ref[...]
    iota = lax.broadcasted_iota(jnp.int32, (_R, _N), 1)
    vals, inds = [], []
    for _t in range(_K):
        p = p_scr[...]
        m = jnp.max(p, axis=1, keepdims=True)
        pos = jnp.min(jnp.where(p == m, iota, _N), axis=1, keepdims=True)
        vals.append(m)
        inds.append(pos)
        p_scr[...] = jnp.where(iota == pos, -jnp.inf, p)
    dist_ref[...] = jnp.concatenate(vals, axis=1)
    idx_ref[...] = jnp.concatenate(inds, axis=1)
    ut_ref[...] = lax.dot_general(xt, awt_ref[...], (((1,), (0,)), ((), ())),
                                  preferred_element_type=jnp.float32,
                                  precision=lax.Precision.HIGHEST)
    vt_ref[...] = lax.dot_general(xt, bwt_ref[...], (((1,), (0,)), ((), ())),
                                  preferred_element_type=jnp.float32,
                                  precision=lax.Precision.HIGHEST)


def _run_topk_proj_b(xt_b, xb_b, xxr_b, xxc_b, awt, bwt):
    nblk = _N // _R
    return pl.pallas_call(
        _topk_proj_body,
        grid=(nblk,),
        in_specs=[
            pl.BlockSpec((_R, _C), lambda r: (r, 0)),
            pl.BlockSpec((_C, _N), lambda r: (0, 0)),
            pl.BlockSpec((1, _N), lambda r: (0, 0)),
            pl.BlockSpec((_R, 1), lambda r: (r, 0)),
            pl.BlockSpec((_C, _DO), lambda r: (0, 0)),
            pl.BlockSpec((_C, _DO), lambda r: (0, 0)),
        ],
        out_specs=[
            pl.BlockSpec((_R, _K), lambda r: (r, 0)),
            pl.BlockSpec((_R, _K), lambda r: (r, 0)),
            pl.BlockSpec((_R, _DO), lambda r: (r, 0)),
            pl.BlockSpec((_R, _DO), lambda r: (r, 0)),
        ],
        out_shape=[
            jax.ShapeDtypeStruct((_N, _K), jnp.int32),
            jax.ShapeDtypeStruct((_N, _K), jnp.float32),
            jax.ShapeDtypeStruct((_N, _DO), jnp.float32),
            jax.ShapeDtypeStruct((_N, _DO), jnp.float32),
        ],
        scratch_shapes=[pltpu.VMEM((_R, _N), jnp.float32)],
    )(xt_b, xb_b, xxr_b, xxc_b, awt, bwt)


def _run_sc_gather_b(vt_b, idx_row):
    mesh = plsc.VectorSubcoreMesh(core_axis_name="c", subcore_axis_name="s")

    def kbody(vt_hbm, idx_hbm, out_hbm):
        def body(i_vmem, o_vmem):
            pltpu.sync_copy(vt_hbm.at[i_vmem.at[0]], o_vmem)

        pltpu.emit_pipeline(
            body,
            grid=(_NK // _G,),
            in_specs=[pl.BlockSpec((1, _G), lambda i: (0, i))],
            out_specs=[pl.BlockSpec((_G, _DO), lambda i: (i, 0))],
            core_axis_name=("c", "s"),
            dimension_semantics=(pltpu.PARALLEL,),
        )(idx_hbm, out_hbm)

    k = pl.kernel(
        kbody,
        out_type=jax.ShapeDtypeStruct((_NK, _DO), jnp.float32),
        mesh=mesh,
        scratch_types=[],
    )
    return k(vt_b, idx_row)


def _stats_body(tmp_ref, ut_ref, d_ref, e_ref, s4_ref, out_ref):
    i = pl.program_id(0)
    u_exp = lax.dot_general(e_ref[...], ut_ref[...], (((1,), (0,)), ((), ())),
                            preferred_element_type=jnp.float32,
                            precision=lax.Precision.HIGHEST)
    d16 = d_ref[...].astype(jnp.bfloat16).astype(jnp.float32)
    y = tmp_ref[...] + u_exp + d16 * s4_ref[...]
    blk = jnp.concatenate([jnp.sum(y, axis=0, keepdims=True),
                           jnp.sum(y * y, axis=0, keepdims=True)], axis=0)

    @pl.when(i == 0)
    def _():
        out_ref[...] = blk

    @pl.when(i > 0)
    def _():
        out_ref[...] += blk


def _run_stats_b(tmp_b, ut_b, dcol_b, e, s4r):
    nblk = _N // _NB
    return pl.pallas_call(
        _stats_body,
        grid=(nblk,),
        in_specs=[
            pl.BlockSpec((_NBK, _DO), lambda i: (i, 0)),
            pl.BlockSpec((_NB, _DO), lambda i: (i, 0)),
            pl.BlockSpec((_NBK, 1), lambda i: (i, 0)),
            pl.BlockSpec((_NBK, _NB), lambda i: (0, 0)),
            pl.BlockSpec((1, _DO), lambda i: (0, 0)),
        ],
        out_specs=pl.BlockSpec((2, _DO), lambda i: (0, 0)),
        out_shape=jax.ShapeDtypeStruct((2, _DO), jnp.float32),
    )(tmp_b, ut_b, dcol_b, e, s4r)


def _emit_body(tmp_ref, ut_ref, d_ref, e_ref, s4_ref, st_ref, g_ref, be_ref,
               out_ref):
    st = st_ref[...]
    mean = st[0:1, :] * (1.0 / (_B * _NK))
    var = st[1:2, :] * (1.0 / (_B * _NK)) - mean * mean
    scale = g_ref[...] * lax.rsqrt(var + 1e-5)
    bias = be_ref[...] - mean * scale
    u_exp = lax.dot_general(e_ref[...], ut_ref[...], (((1,), (0,)), ((), ())),
                            preferred_element_type=jnp.float32,
                            precision=lax.Precision.HIGHEST)
    d16 = d_ref[...].astype(jnp.bfloat16).astype(jnp.float32)
    y = tmp_ref[...] + u_exp + d16 * s4_ref[...]
    z = y * scale + bias
    z = jnp.where(z > 0, z, 0.2 * z)
    out_ref[...] = z.T


def _run_emit_b(tmp_b, ut_b, dcol_b, e, s4r, stats, gr, br):
    nblk = _N // _NB
    return pl.pallas_call(
        _emit_body,
        grid=(nblk,),
        in_specs=[
            pl.BlockSpec((_NBK, _DO), lambda r: (r, 0)),
            pl.BlockSpec((_NB, _DO), lambda r: (r, 0)),
            pl.BlockSpec((_NBK, 1), lambda r: (r, 0)),
            pl.BlockSpec((_NBK, _NB), lambda r: (0, 0)),
            pl.BlockSpec((1, _DO), lambda r: (0, 0)),
            pl.BlockSpec((2, _DO), lambda r: (0, 0)),
            pl.BlockSpec((1, _DO), lambda r: (0, 0)),
            pl.BlockSpec((1, _DO), lambda r: (0, 0)),
        ],
        out_specs=pl.BlockSpec((_DO, _NBK), lambda r: (0, r)),
        out_shape=jax.ShapeDtypeStruct((_DO, _NK), jnp.float32),
    )(tmp_b, ut_b, dcol_b, e, s4r, stats, gr, br)


def kernel(features, W, gamma, beta):
    x = jnp.squeeze(features, -1)
    xT = jnp.swapaxes(x, 1, 2)
    w = W.reshape(_DO, 4 * _C)
    awt = (w[:, :_C] - w[:, 2 * _C:3 * _C]).T
    bwt = (w[:, _C:2 * _C] + w[:, 2 * _C:3 * _C]).T
    w4bits = lax.bitcast_convert_type(w[:, 3 * _C:], jnp.int32)
    w4r = w4bits + jnp.int32(0x7FFF) + jnp.bitwise_and(
        lax.shift_right_logical(w4bits, 16), jnp.int32(1))
    w4b = lax.bitcast_convert_type(
        jnp.bitwise_and(w4r, jnp.int32(-65536)), jnp.float32)
    s4r = jnp.sum(w4b, axis=1).reshape(1, _DO)

    rows = lax.broadcasted_iota(jnp.int32, (_NBK, _NB), 0) // _K
    cols = lax.broadcasted_iota(jnp.int32, (_NBK, _NB), 1)
    e = (rows == cols).astype(jnp.float32)

    sq = _run_sq(x)

    per_b = []
    for b in range(_B):
        idx_b, dist_b, ut_b, vt_b = _run_topk_proj_b(
            xT[b], x[b], sq[b], jnp.swapaxes(sq[b], 0, 1), awt, bwt)
        tmp_b = _run_sc_gather_b(vt_b, idx_b.reshape(1, _NK))
        per_b.append((tmp_b, ut_b, dist_b.reshape(_NK, 1)))

    parts = [_run_stats_b(t, u, d, e, s4r) for (t, u, d) in per_b]
    stats = parts[0] + parts[1] + parts[2] + parts[3]

    gr = gamma.reshape(1, _DO)
    br = beta.reshape(1, _DO)
    outs = [_run_emit_b(t, u, d, e, s4r, stats, gr, br)
            for (t, u, d) in per_b]
    return jnp.stack(outs, axis=0).reshape(_B, _DO, _N, _K)
